# Initial kernel scaffold; baseline (speedup 1.0000x reference)
import jax
import jax.numpy as jnp
from jax.experimental import pallas as pl


def kernel(x, edge_index, edge_weight, W):
    raise NotImplementedError("write your pallas kernel here")


# speedup vs baseline: 4.0615x; 4.0615x over previous
import functools

import jax
import jax.numpy as jnp
from jax import lax
from jax.experimental import pallas as pl
from jax.experimental.pallas import tpu as pltpu
from jax.experimental.pallas import tpu_sc as plsc

N_NODES = 10000
N_EDGES = 320000
D = 128

NC = 2
NS = 16
L = 16
NW = NC * NS

EDGES_PER_TILE = N_EDGES // NW
CHUNK = 80
NCHUNK = EDGES_PER_TILE // CHUNK
N_PAD = 10240
ROWS_PER_TILE = N_PAD // NS
ZROWS = 128


def _matmul(x, W):
    def mm_kernel(x_ref, w_ref, o_ref):
        o_ref[...] = jnp.dot(x_ref[...], w_ref[...],
                             preferred_element_type=jnp.float32)

    return pl.pallas_call(
        mm_kernel,
        grid=(10,),
        in_specs=[
            pl.BlockSpec((1000, D), lambda i: (i, 0)),
            pl.BlockSpec((D, D), lambda i: (0, 0)),
        ],
        out_specs=pl.BlockSpec((1000, D), lambda i: (i, 0)),
        out_shape=jax.ShapeDtypeStruct((N_NODES, D), jnp.float32),
    )(x, W)


_SC_MESH = plsc.VectorSubcoreMesh(
    core_axis_name="c", subcore_axis_name="s", num_cores=NC, num_subcores=NS)


@functools.partial(
    pl.kernel,
    mesh=_SC_MESH,
    out_type=jax.ShapeDtypeStruct((NC, N_PAD, D), jnp.float32),
    scratch_types=[
        pltpu.VMEM((CHUNK,), jnp.int32),
        pltpu.VMEM((CHUNK,), jnp.int32),
        pltpu.VMEM((CHUNK,), jnp.float32),
        pltpu.VMEM((CHUNK, D), jnp.float32),
        pltpu.VMEM((ZROWS, D), jnp.float32),
        pltpu.VMEM_SHARED((N_PAD, D), jnp.float32),
        pltpu.SemaphoreType.DMA,
    ],
)
def _sc_aggregate(support_hbm, src_hbm, dst_hbm, w_hbm, out_hbm,
                  src_v, dst_v, w_v, rows_v, z_v, accum, sem):
    c = lax.axis_index("c")
    s = lax.axis_index("s")

    def zero_row(i, _):
        for cc in range(D // L):
            z_v[i, pl.ds(cc * L, L)] = jnp.zeros((L,), jnp.float32)
        return 0
    lax.fori_loop(0, ZROWS, zero_row, 0)
    row0 = s * ROWS_PER_TILE
    for b in range(ROWS_PER_TILE // ZROWS):
        pltpu.sync_copy(z_v, accum.at[pl.ds(row0 + b * ZROWS, ZROWS)])
    plsc.subcore_barrier()

    base_e = (c * NS + s) * EDGES_PER_TILE

    def edge_chunk(i, _):
        off = base_e + i * CHUNK
        pltpu.sync_copy(src_hbm.at[pl.ds(off, CHUNK)], src_v)
        pltpu.sync_copy(dst_hbm.at[pl.ds(off, CHUNK)], dst_v)
        pltpu.sync_copy(w_hbm.at[pl.ds(off, CHUNK)], w_v)
        pltpu.async_copy(support_hbm.at[src_v], rows_v, sem).wait()

        def scale_group(g, _):
            wv = w_v[pl.ds(g * L, L)]
            for j in range(L):
                wvec = jnp.full((L,), wv[j], jnp.float32)
                r = g * L + j
                for cc in range(D // L):
                    sl = pl.ds(cc * L, L)
                    rows_v[r, sl] = rows_v[r, sl] * wvec
            return 0
        lax.fori_loop(0, CHUNK // L, scale_group, 0)

        pltpu.sync_copy(rows_v, accum.at[dst_v], add=True)
        return 0
    lax.fori_loop(0, NCHUNK, edge_chunk, 0)
    plsc.subcore_barrier()

    pltpu.sync_copy(accum.at[pl.ds(row0, ROWS_PER_TILE)],
                    out_hbm.at[c, pl.ds(row0, ROWS_PER_TILE)])


def _add_relu(partials):
    def ar_kernel(p_ref, o_ref):
        o_ref[...] = jnp.maximum(p_ref[0] + p_ref[1], 0.0)

    return pl.pallas_call(
        ar_kernel,
        grid=(10,),
        in_specs=[pl.BlockSpec((NC, 1000, D), lambda i: (0, i, 0))],
        out_specs=pl.BlockSpec((1000, D), lambda i: (i, 0)),
        out_shape=jax.ShapeDtypeStruct((N_NODES, D), jnp.float32),
    )(partials)


def kernel(x, edge_index, edge_weight, W):
    support = _matmul(x, W)
    dst = edge_index[0].astype(jnp.int32)
    src = edge_index[1].astype(jnp.int32)
    partials = _sc_aggregate(support, src, dst, edge_weight)
    return _add_relu(partials[:, :N_NODES])
